# baseline (speedup 1.0000x reference)
import functools

import jax
import jax.numpy as jnp
from jax import lax
from jax.experimental import pallas as pl
from jax.experimental.pallas import tpu as pltpu
from jax.experimental.pallas import tpu_sc as plsc

_B = 16384
_DIM = 64
_NREL = 32
_LAMBDA = 1e-05

_NW = 32
_ROWS = 3 * _B
_BPW = _ROWS // _NW
_CHUNK = 128
_NCHUNK = _BPW // _CHUNK

_sc_mesh = plsc.VectorSubcoreMesh(core_axis_name="c", subcore_axis_name="s")


@functools.partial(
    pl.kernel,
    out_type=jax.ShapeDtypeStruct((_ROWS, _DIM), jnp.float32),
    mesh=_sc_mesh,
    scratch_types=[
        pltpu.VMEM((_BPW,), jnp.int32),
        pltpu.VMEM((_BPW, _DIM), jnp.float32),
        pltpu.SemaphoreType.DMA,
    ],
    compiler_params=pltpu.CompilerParams(use_tc_tiling_on_sc=False),
)
def _sc_gather(table_hbm, idx_hbm, out_hbm, idx_v, rows_v, sem):
    wid = lax.axis_index("s") * 2 + lax.axis_index("c")
    base = wid * _BPW
    pltpu.sync_copy(idx_hbm.at[pl.ds(base, _BPW)], idx_v)
    copies = [
        pltpu.async_copy(
            table_hbm.at[idx_v.at[pl.ds(j * _CHUNK, _CHUNK)]],
            rows_v.at[pl.ds(j * _CHUNK, _CHUNK)],
            sem,
        )
        for j in range(_NCHUNK)
    ]
    for c in copies:
        c.wait()
    pltpu.sync_copy(rows_v, out_hbm.at[pl.ds(base, _BPW)])


_BB = 1024
_NB = _B // _BB
_KD = _NREL * _DIM


def _tc_body(g_ref, rcol_ref, rel_ref, wt_ref, out_ref, acc):
    i = pl.program_id(0)

    @pl.when(i == 0)
    def _init():
        for j in range(5):
            acc[j] = 0.0

    rcol = rcol_ref[...]
    rel = rel_ref[...]
    wt = wt_ref[...]

    onehot = (rcol == lax.broadcasted_iota(jnp.int32, (_BB, _NREL), 1)
              ).astype(jnp.float32)
    r_emb = jnp.dot(onehot, rel, preferred_element_type=jnp.float32)

    gid = lax.broadcasted_iota(jnp.int32, (_BB, _KD), 1) // _DIM

    def proj(x):
        p = jnp.dot(x, wt, precision=lax.Precision.HIGHEST,
                    preferred_element_type=jnp.float32)
        a = jnp.where(rcol == gid, p, 0.0)
        w = _KD // 2
        while w >= _DIM:
            a = a[:, :w] + a[:, w:]
            w //= 2
        return a

    ph = proj(g_ref[0])
    pp = proj(g_ref[1])
    pn = proj(g_ref[2])

    anchor = ph + r_emb
    dpos = anchor - pp
    dneg = anchor - pn
    pos_s = jnp.sum(dpos * dpos, axis=1, keepdims=True)
    neg_s = jnp.sum(dneg * dneg, axis=1, keepdims=True)
    y = neg_s - pos_s
    term = jnp.maximum(-y, 0.0) + jnp.log(1.0 + jnp.exp(-jnp.abs(y)))

    acc[0] += jnp.sum(term)
    acc[1] += jnp.sum(ph * ph)
    acc[2] += jnp.sum(r_emb * r_emb)
    acc[3] += jnp.sum(pp * pp)
    acc[4] += jnp.sum(pn * pn)

    @pl.when(i == _NB - 1)
    def _emit():
        kg = acc[0] / _B
        l2 = (acc[1] + acc[2] + acc[3] + acc[4]) / (2.0 * _B)
        out_ref[0, 0] = kg + _LAMBDA * l2


_tc_call = pl.pallas_call(
    _tc_body,
    grid=(_NB,),
    in_specs=[
        pl.BlockSpec((3, _BB, _DIM), lambda i: (0, i, 0)),
        pl.BlockSpec((_BB, 1), lambda i: (i, 0)),
        pl.BlockSpec((_NREL, _DIM), lambda i: (0, 0)),
        pl.BlockSpec((_DIM, _KD), lambda i: (0, 0)),
    ],
    out_specs=pl.BlockSpec((1, 1), lambda i: (0, 0), memory_space=pltpu.SMEM),
    out_shape=jax.ShapeDtypeStruct((1, 1), jnp.float32),
    scratch_shapes=[pltpu.SMEM((8,), jnp.float32)],
)


def kernel(entity_emb, relation_emb, transfer_matrix, h, r, pos_t, neg_t):
    idx = jnp.concatenate([h, pos_t, neg_t]).astype(jnp.int32)
    gathered = _sc_gather(entity_emb, idx)
    g3 = gathered.reshape(3, _B, _DIM)
    rcol = r.astype(jnp.int32).reshape(_B, 1)
    wt = jnp.transpose(transfer_matrix, (1, 0, 2)).reshape(_DIM, _KD)
    loss = _tc_call(g3, rcol, relation_emb, wt)
    return loss[0, 0]


# speedup vs baseline: 1.3283x; 1.3283x over previous
import functools

import jax
import jax.numpy as jnp
from jax import lax
from jax.experimental import pallas as pl
from jax.experimental.pallas import tpu as pltpu
from jax.experimental.pallas import tpu_sc as plsc

_B = 16384
_DIM = 64
_NREL = 32
_LAMBDA = 1e-05

_NW = 32
_ROWS = 3 * _B
_BPW = _ROWS // _NW
_CHUNK = 128
_NCHUNK = _BPW // _CHUNK

_sc_mesh = plsc.VectorSubcoreMesh(core_axis_name="c", subcore_axis_name="s")


@functools.partial(
    pl.kernel,
    out_type=jax.ShapeDtypeStruct((_ROWS, _DIM), jnp.float32),
    mesh=_sc_mesh,
    scratch_types=[
        pltpu.VMEM((_BPW,), jnp.int32),
        pltpu.VMEM((_BPW, _DIM), jnp.float32),
        pltpu.SemaphoreType.DMA,
    ],
    compiler_params=pltpu.CompilerParams(use_tc_tiling_on_sc=False),
)
def _sc_gather(table_hbm, idx_hbm, out_hbm, idx_v, rows_v, sem):
    wid = lax.axis_index("s") * 2 + lax.axis_index("c")
    base = wid * _BPW
    pltpu.sync_copy(idx_hbm.at[pl.ds(base, _BPW)], idx_v)
    copies = [
        pltpu.async_copy(
            table_hbm.at[idx_v.at[pl.ds(j * _CHUNK, _CHUNK)]],
            rows_v.at[pl.ds(j * _CHUNK, _CHUNK)],
            sem,
        )
        for j in range(_NCHUNK)
    ]
    for c in copies:
        c.wait()
    pltpu.sync_copy(rows_v, out_hbm.at[pl.ds(base, _BPW)])


_BB = 1024
_NB = _B // _BB
_KD = _NREL * _DIM


def _tc_body(g_ref, rcol_ref, rel_ref, wt_ref, out_ref, acc):
    i = pl.program_id(0)

    @pl.when(i == 0)
    def _init():
        for j in range(5):
            acc[j] = 0.0

    rcol = rcol_ref[...]
    rel = rel_ref[...]
    wt = wt_ref[...]

    onehot = (rcol == lax.broadcasted_iota(jnp.int32, (_BB, _NREL), 1)
              ).astype(jnp.float32)
    r_emb = jnp.dot(onehot, rel, preferred_element_type=jnp.float32)

    gid = lax.broadcasted_iota(jnp.int32, (_BB, _KD), 1) // _DIM

    def proj(x):
        p = jnp.dot(x, wt, preferred_element_type=jnp.float32)
        a = jnp.where(rcol == gid, p, 0.0)
        w = _KD // 2
        while w >= _DIM:
            a = a[:, :w] + a[:, w:]
            w //= 2
        return a

    ph = proj(g_ref[0])
    pp = proj(g_ref[1])
    pn = proj(g_ref[2])

    anchor = ph + r_emb
    dpos = anchor - pp
    dneg = anchor - pn
    pos_s = jnp.sum(dpos * dpos, axis=1, keepdims=True)
    neg_s = jnp.sum(dneg * dneg, axis=1, keepdims=True)
    y = neg_s - pos_s
    term = jnp.maximum(-y, 0.0) + jnp.log(1.0 + jnp.exp(-jnp.abs(y)))

    acc[0] += jnp.sum(term)
    acc[1] += jnp.sum(ph * ph)
    acc[2] += jnp.sum(r_emb * r_emb)
    acc[3] += jnp.sum(pp * pp)
    acc[4] += jnp.sum(pn * pn)

    @pl.when(i == _NB - 1)
    def _emit():
        kg = acc[0] / _B
        l2 = (acc[1] + acc[2] + acc[3] + acc[4]) / (2.0 * _B)
        out_ref[0, 0] = kg + _LAMBDA * l2


_tc_call = pl.pallas_call(
    _tc_body,
    grid=(_NB,),
    in_specs=[
        pl.BlockSpec((3, _BB, _DIM), lambda i: (0, i, 0)),
        pl.BlockSpec((_BB, 1), lambda i: (i, 0)),
        pl.BlockSpec((_NREL, _DIM), lambda i: (0, 0)),
        pl.BlockSpec((_DIM, _KD), lambda i: (0, 0)),
    ],
    out_specs=pl.BlockSpec((1, 1), lambda i: (0, 0), memory_space=pltpu.SMEM),
    out_shape=jax.ShapeDtypeStruct((1, 1), jnp.float32),
    scratch_shapes=[pltpu.SMEM((8,), jnp.float32)],
)


def kernel(entity_emb, relation_emb, transfer_matrix, h, r, pos_t, neg_t):
    idx = jnp.concatenate([h, pos_t, neg_t]).astype(jnp.int32)
    gathered = _sc_gather(entity_emb, idx)
    g3 = gathered.reshape(3, _B, _DIM)
    rcol = r.astype(jnp.int32).reshape(_B, 1)
    wt = jnp.transpose(transfer_matrix, (1, 0, 2)).reshape(_DIM, _KD)
    loss = _tc_call(g3, rcol, relation_emb, wt)
    return loss[0, 0]
